# Initial kernel scaffold; baseline (speedup 1.0000x reference)
import jax
import jax.numpy as jnp
from jax.experimental import pallas as pl


def kernel(x, edge_index, params):
    raise NotImplementedError("write your pallas kernel here")


# speedup vs baseline: 1.3860x; 1.3860x over previous
import functools

import jax
import jax.numpy as jnp
from jax import lax
from jax.experimental import pallas as pl
from jax.experimental.pallas import tpu as pltpu
from jax.experimental.pallas import tpu_sc as plsc

N = 2048
E = 65536
DEPTH = 4


_SLAB = 64
_HALF = 32
_CH = 8192


def _build_t_sc(rows, cols, zblock):
    info = plsc.get_sparse_core_info()
    nc = info.num_cores

    mesh = plsc.VectorSubcoreMesh(core_axis_name="c", subcore_axis_name="s")

    @functools.partial(
        pl.kernel,
        mesh=mesh,
        out_type=jax.ShapeDtypeStruct((N * N,), jnp.float32),
        scratch_types=[
            pltpu.VMEM((_HALF * N,), jnp.float32),
            pltpu.VMEM((_CH,), jnp.int32),
            pltpu.VMEM((_CH,), jnp.int32),
        ],
        compiler_params=pltpu.CompilerParams(needs_layout_passes=False),
    )
    def k(rows_hbm, cols_hbm, z_hbm, out_hbm, acc, rv, cv):
        wid = lax.axis_index("s") * nc + lax.axis_index("c")
        ones16 = jnp.ones((16,), jnp.float32)
        for p in range(_SLAB // _HALF):
            base = wid * _SLAB + p * _HALF
            pltpu.sync_copy(z_hbm, acc)
            for ch in range(E // _CH):
                pltpu.sync_copy(rows_hbm.at[pl.ds(ch * _CH, _CH)], rv)
                pltpu.sync_copy(cols_hbm.at[pl.ds(ch * _CH, _CH)], cv)

                def body(g, carry):
                    r16 = rv[pl.ds(g * 16, 16)]
                    c16 = cv[pl.ds(g * 16, 16)]
                    lr = c16 - base
                    m = (lr >= 0) & (lr < _HALF)
                    lr = jnp.where(m, lr, 0)
                    plsc.addupdate_scatter(acc, [lr * N + r16], ones16, mask=m)
                    return carry

                lax.fori_loop(0, _CH // 16, body, 0)
            pltpu.sync_copy(acc, out_hbm.at[pl.ds(base * N, _HALF * N)])

    return k(rows, cols, zblock).reshape(N, N)


_NN = (((1,), (0,)), ((), ()))
_TN = (((0,), (0,)), ((), ()))


def _b(x):
    return x.astype(jnp.bfloat16)


def _bfdot(a, b, dims=_NN):
    return lax.dot_general(_b(a), _b(b), dims,
                           preferred_element_type=jnp.float32)


def _split3(x):
    h1 = _b(x).astype(jnp.float32)
    r1 = x - h1
    h2 = _b(r1).astype(jnp.float32)
    r2 = r1 - h2
    return h1, h2, r2


def _exact_dot(sel, m, dims):
    selb = _b(sel)
    out = None
    for part in _split3(m):
        d = lax.dot_general(selb, _b(part), dims,
                            preferred_element_type=jnp.float32)
        out = d if out is None else out + d
    return out


def _eye(n):
    r = lax.broadcasted_iota(jnp.int32, (n, n), 0)
    c = lax.broadcasted_iota(jnp.int32, (n, n), 1)
    return jnp.where(r == c, 1.0, 0.0).astype(jnp.float32)


def _gcn_body(t_mat, xw, b_row):
    n = t_mat.shape[0]
    ey = _eye(n)
    dvec = jnp.sum(t_mat * ey, axis=1, keepdims=True)
    fill = jnp.where(dvec == 0.0, 2.0, 0.0)
    t2 = t_mat + ey * fill
    deg = jnp.sum(t2, axis=1, keepdims=True)
    dinv = jnp.where(deg > 0.0,
                     1.0 / jnp.sqrt(jnp.maximum(deg, 1e-12)), 0.0)
    ant = (jnp.transpose(dinv, (1, 0)) * t2) * dinv
    return _bfdot(ant, xw) + b_row


def _proj_kernel(x_ref, w_ref, o_ref):
    o_ref[...] = _bfdot(x_ref[...], w_ref[...])


def _gcn0_kernel(t_ref, xw_ref, b_ref, o_ref):
    y = _gcn_body(t_ref[...], xw_ref[...], b_ref[...])
    o_ref[...] = jnp.maximum(y, 0.0)


def _transition_kernel(t_ref, x_ref, pw_ref, w_ref, b_ref,
                       tn_ref, xn_ref, rt_ref, *, n, kk):
    t_prev = t_ref[...]
    x_prev = x_ref[...]
    pw = pw_ref[...]

    ey = _eye(n)
    t_tilde = t_prev * (1.0 - ey) + ey

    s_raw = _bfdot(x_prev, jnp.transpose(pw, (1, 0)))
    s_col = jnp.tanh(s_raw / jnp.sqrt(jnp.sum(pw * pw)))
    s_row = jnp.transpose(s_col, (1, 0))
    i_col = lax.broadcasted_iota(jnp.int32, (n, 1), 0)
    i_row = lax.broadcasted_iota(jnp.int32, (1, n), 1)
    gt = (s_row > s_col) | ((s_row == s_col) & (i_row < i_col))
    rank = jnp.sum(jnp.where(gt, 1.0, 0.0), axis=1, keepdims=True)
    r_row = lax.broadcasted_iota(jnp.int32, (1, kk), 1).astype(jnp.float32)
    rt = jnp.where(rank == r_row, 1.0, 0.0)

    vals = _exact_dot(rt, s_col, _TN)
    xp = _exact_dot(rt, x_prev, _TN) * vals

    c1 = _bfdot(t_tilde, rt)
    c2 = _bfdot(t_tilde, c1)
    c3 = _exact_dot(rt, c2, _TN)
    eyk = _eye(kk)
    t_next = c3 * (1.0 - eyk)

    t2 = t_next + 2.0 * eyk
    deg = jnp.sum(t2, axis=1, keepdims=True)
    dinv = jnp.where(deg > 0.0,
                     1.0 / jnp.sqrt(jnp.maximum(deg, 1e-12)), 0.0)
    ant = (jnp.transpose(dinv, (1, 0)) * t2) * dinv
    xw = _bfdot(xp, w_ref[...])
    y = _bfdot(ant, xw) + b_ref[...]

    tn_ref[...] = t_next
    xn_ref[...] = jnp.maximum(y, 0.0)
    rt_ref[...] = rt


def _up_kernel(t_ref, res_ref, x_ref, rt_ref, w_ref, b_ref, o_ref,
               *, last):
    up = _exact_dot(rt_ref[...], x_ref[...], _NN)
    x = res_ref[...] + up
    xw = _bfdot(x, w_ref[...])
    y = _gcn_body(t_ref[...], xw, b_ref[...])
    if last:
        ss = jnp.sum(y * y, axis=1, keepdims=True)
        y = y / jnp.maximum(jnp.sqrt(jnp.maximum(ss, 1e-12)), 1e-3)
    else:
        y = jnp.maximum(y, 0.0)
    o_ref[...] = y


def _call(fn, out_shapes, *args, **static):
    f = functools.partial(fn, **static) if static else fn
    return pl.pallas_call(
        f,
        out_shape=out_shapes,
    )(*args)


def kernel(x, edge_index, params):
    f32 = jnp.float32
    rows = edge_index[0].astype(jnp.int32)
    cols = edge_index[1].astype(jnp.int32)
    zblock = jnp.zeros((_HALF * N,), f32)

    xw0 = _call(_proj_kernel, jax.ShapeDtypeStruct((N, 16), f32),
                x, params["W_down"][0])

    t0 = _build_t_sc(rows, cols, zblock)

    b0 = params["b_down"][0].reshape(1, -1)
    x_cur = _call(_gcn0_kernel, jax.ShapeDtypeStruct((N, 16), f32),
                  t0, xw0, b0)

    sizes = [N // (2 ** i) for i in range(DEPTH + 1)]
    ts, xs, rts = [t0], [x_cur], []
    for i in range(1, DEPTH + 1):
        n, kk = sizes[i - 1], sizes[i]
        pw = params["pool_w"][i - 1].reshape(1, -1)
        wd = params["W_down"][i]
        bd = params["b_down"][i].reshape(1, -1)
        outs = (jax.ShapeDtypeStruct((kk, kk), f32),
                jax.ShapeDtypeStruct((kk, 16), f32),
                jax.ShapeDtypeStruct((n, kk), f32))
        t_next, x_cur, rt = _call(_transition_kernel, outs,
                                  ts[-1], xs[-1], pw, wd, bd, n=n, kk=kk)
        ts.append(t_next)
        xs.append(x_cur)
        rts.append(rt)

    for i in range(DEPTH):
        j = DEPTH - 1 - i
        wu = params["W_up"][i]
        bu = params["b_up"][i].reshape(1, -1)
        out_ch = wu.shape[1]
        x_cur = _call(_up_kernel,
                      jax.ShapeDtypeStruct((sizes[j], out_ch), f32),
                      ts[j], xs[j], x_cur, rts[j], wu, bu,
                      last=(i == DEPTH - 1))

    return x_cur


# speedup vs baseline: 1.4386x; 1.0379x over previous
import functools

import jax
import jax.numpy as jnp
from jax import lax
from jax.experimental import pallas as pl
from jax.experimental.pallas import tpu as pltpu
from jax.experimental.pallas import tpu_sc as plsc

N = 2048
E = 65536
DEPTH = 4


_SLAB = 64
_HALF = 32
_CH = 8192


def _build_t_sc(rows, cols, zblock):
    info = plsc.get_sparse_core_info()
    nc = info.num_cores

    mesh = plsc.VectorSubcoreMesh(core_axis_name="c", subcore_axis_name="s")

    @functools.partial(
        pl.kernel,
        mesh=mesh,
        out_type=jax.ShapeDtypeStruct((N * N,), jnp.float32),
        scratch_types=[
            pltpu.VMEM((_HALF * N,), jnp.float32),
            pltpu.VMEM((_CH,), jnp.int32),
            pltpu.VMEM((_CH,), jnp.int32),
        ],
        compiler_params=pltpu.CompilerParams(needs_layout_passes=False),
    )
    def k(rows_hbm, cols_hbm, z_hbm, out_hbm, acc, rv, cv):
        wid = lax.axis_index("s") * nc + lax.axis_index("c")
        ones16 = jnp.ones((16,), jnp.float32)
        for p in range(_SLAB // _HALF):
            base = wid * _SLAB + p * _HALF
            pltpu.sync_copy(z_hbm, acc)
            for ch in range(E // _CH):
                pltpu.sync_copy(rows_hbm.at[pl.ds(ch * _CH, _CH)], rv)
                pltpu.sync_copy(cols_hbm.at[pl.ds(ch * _CH, _CH)], cv)

                def body(g, carry):
                    r16 = rv[pl.ds(g * 16, 16)]
                    c16 = cv[pl.ds(g * 16, 16)]
                    lr = c16 - base
                    m = (lr >= 0) & (lr < _HALF)
                    lr = jnp.where(m, lr, 0)
                    plsc.addupdate_scatter(acc, [lr * N + r16], ones16, mask=m)
                    return carry

                lax.fori_loop(0, _CH // 16, body, 0)
            pltpu.sync_copy(acc, out_hbm.at[pl.ds(base * N, _HALF * N)])

    return k(rows, cols, zblock).reshape(N, N)


_NN = (((1,), (0,)), ((), ()))
_TN = (((0,), (0,)), ((), ()))


def _b(x):
    return x.astype(jnp.bfloat16)


def _bfdot(a, b, dims=_NN):
    return lax.dot_general(_b(a), _b(b), dims,
                           preferred_element_type=jnp.float32)


def _split3(x):
    h1 = _b(x).astype(jnp.float32)
    r1 = x - h1
    h2 = _b(r1).astype(jnp.float32)
    r2 = r1 - h2
    return h1, h2, r2


def _exact_dot(sel, m, dims):
    selb = _b(sel)
    out = None
    for part in _split3(m):
        d = lax.dot_general(selb, _b(part), dims,
                            preferred_element_type=jnp.float32)
        out = d if out is None else out + d
    return out


def _eye(n):
    r = lax.broadcasted_iota(jnp.int32, (n, n), 0)
    c = lax.broadcasted_iota(jnp.int32, (n, n), 1)
    return jnp.where(r == c, 1.0, 0.0).astype(jnp.float32)


def _gcn_body(t_mat, xw, b_row):
    n = t_mat.shape[0]
    ey = _eye(n)
    dvec = jnp.sum(t_mat * ey, axis=1, keepdims=True)
    fill = jnp.where(dvec == 0.0, 2.0, 0.0)
    t2 = t_mat + ey * fill
    deg = jnp.sum(t2, axis=1, keepdims=True)
    dinv = jnp.where(deg > 0.0,
                     1.0 / jnp.sqrt(jnp.maximum(deg, 1e-12)), 0.0)
    ant = (jnp.transpose(dinv, (1, 0)) * t2) * dinv
    return _bfdot(ant, xw) + b_row


def _proj_kernel(x_ref, w_ref, o_ref):
    o_ref[...] = _bfdot(x_ref[...], w_ref[...])


def _gcn0_kernel(t_ref, xw_ref, b_ref, o_ref):
    y = _gcn_body(t_ref[...], xw_ref[...], b_ref[...])
    o_ref[...] = jnp.maximum(y, 0.0)


def _transition_kernel(t_ref, x_ref, pw_ref, w_ref, b_ref,
                       tn_ref, xn_ref, rt_ref, *, n, kk):
    t_prev = t_ref[...]
    x_prev = x_ref[...]
    pw = pw_ref[...]

    ey = _eye(n)
    t_tilde = t_prev * (1.0 - ey) + ey

    s_raw = _bfdot(x_prev, jnp.transpose(pw, (1, 0)))
    s_col = jnp.tanh(s_raw / jnp.sqrt(jnp.sum(pw * pw)))
    s_row = jnp.transpose(s_col, (1, 0))
    i_col = lax.broadcasted_iota(jnp.int32, (n, 1), 0)
    i_row = lax.broadcasted_iota(jnp.int32, (1, n), 1)
    gt = (s_row > s_col) | ((s_row == s_col) & (i_row < i_col))
    rank = jnp.sum(jnp.where(gt, 1.0, 0.0), axis=1, keepdims=True)
    r_row = lax.broadcasted_iota(jnp.int32, (1, kk), 1).astype(jnp.float32)
    rt = jnp.where(rank == r_row, 1.0, 0.0)

    vals = _exact_dot(rt, s_col, _TN)
    xp = _exact_dot(rt, x_prev, _TN) * vals

    tb = _b(t_tilde)
    rtb = _b(rt)
    bsel = lax.dot_general(rtb, tb, _TN,
                           preferred_element_type=jnp.float32)
    csel = lax.dot_general(tb, rtb, _NN,
                           preferred_element_type=jnp.float32)
    c3 = _bfdot(bsel, csel)
    eyk = _eye(kk)
    t_next = c3 * (1.0 - eyk)

    t2 = t_next + 2.0 * eyk
    deg = jnp.sum(t2, axis=1, keepdims=True)
    dinv = jnp.where(deg > 0.0,
                     1.0 / jnp.sqrt(jnp.maximum(deg, 1e-12)), 0.0)
    ant = (jnp.transpose(dinv, (1, 0)) * t2) * dinv
    xw = _bfdot(xp, w_ref[...])
    y = _bfdot(ant, xw) + b_ref[...]

    tn_ref[...] = t_next
    xn_ref[...] = jnp.maximum(y, 0.0)
    rt_ref[...] = rt


def _up_kernel(t_ref, res_ref, x_ref, rt_ref, w_ref, b_ref, o_ref,
               *, last):
    up = _exact_dot(rt_ref[...], x_ref[...], _NN)
    x = res_ref[...] + up
    xw = _bfdot(x, w_ref[...])
    y = _gcn_body(t_ref[...], xw, b_ref[...])
    if last:
        ss = jnp.sum(y * y, axis=1, keepdims=True)
        y = y / jnp.maximum(jnp.sqrt(jnp.maximum(ss, 1e-12)), 1e-3)
    else:
        y = jnp.maximum(y, 0.0)
    o_ref[...] = y


def _call(fn, out_shapes, *args, **static):
    f = functools.partial(fn, **static) if static else fn
    return pl.pallas_call(
        f,
        out_shape=out_shapes,
    )(*args)


def kernel(x, edge_index, params):
    f32 = jnp.float32
    rows = edge_index[0].astype(jnp.int32)
    cols = edge_index[1].astype(jnp.int32)
    zblock = jnp.zeros((_HALF * N,), f32)

    xw0 = _call(_proj_kernel, jax.ShapeDtypeStruct((N, 16), f32),
                x, params["W_down"][0])

    t0 = _build_t_sc(rows, cols, zblock)

    b0 = params["b_down"][0].reshape(1, -1)
    x_cur = _call(_gcn0_kernel, jax.ShapeDtypeStruct((N, 16), f32),
                  t0, xw0, b0)

    sizes = [N // (2 ** i) for i in range(DEPTH + 1)]
    ts, xs, rts = [t0], [x_cur], []
    for i in range(1, DEPTH + 1):
        n, kk = sizes[i - 1], sizes[i]
        pw = params["pool_w"][i - 1].reshape(1, -1)
        wd = params["W_down"][i]
        bd = params["b_down"][i].reshape(1, -1)
        outs = (jax.ShapeDtypeStruct((kk, kk), f32),
                jax.ShapeDtypeStruct((kk, 16), f32),
                jax.ShapeDtypeStruct((n, kk), f32))
        t_next, x_cur, rt = _call(_transition_kernel, outs,
                                  ts[-1], xs[-1], pw, wd, bd, n=n, kk=kk)
        ts.append(t_next)
        xs.append(x_cur)
        rts.append(rt)

    for i in range(DEPTH):
        j = DEPTH - 1 - i
        wu = params["W_up"][i]
        bu = params["b_up"][i].reshape(1, -1)
        out_ch = wu.shape[1]
        x_cur = _call(_up_kernel,
                      jax.ShapeDtypeStruct((sizes[j], out_ch), f32),
                      ts[j], xs[j], x_cur, rts[j], wu, bu,
                      last=(i == DEPTH - 1))

    return x_cur


# speedup vs baseline: 1.5540x; 1.0802x over previous
import functools

import jax
import jax.numpy as jnp
from jax import lax
from jax.experimental import pallas as pl
from jax.experimental.pallas import tpu as pltpu
from jax.experimental.pallas import tpu_sc as plsc

N = 2048
E = 65536
DEPTH = 4


_SLAB = 64
_HALF = 32
_CH = 8192


def _flat_kernel(r_ref, c_ref, o_ref):
    o_ref[...] = c_ref[...] * N + r_ref[...]


def _build_t_sc(flat, zblock):
    info = plsc.get_sparse_core_info()
    nc = info.num_cores

    mesh = plsc.VectorSubcoreMesh(core_axis_name="c", subcore_axis_name="s")

    @functools.partial(
        pl.kernel,
        mesh=mesh,
        out_type=jax.ShapeDtypeStruct((N * N,), jnp.float32),
        scratch_types=[
            pltpu.VMEM((_HALF * N,), jnp.float32),
            pltpu.VMEM((_CH,), jnp.int32),
        ],
        compiler_params=pltpu.CompilerParams(needs_layout_passes=False),
    )
    def k(flat_hbm, z_hbm, out_hbm, acc, fv):
        wid = lax.axis_index("s") * nc + lax.axis_index("c")
        ones16 = jnp.ones((16,), jnp.float32)
        for p in range(_SLAB // _HALF):
            base = (wid * _SLAB + p * _HALF) * N
            pltpu.sync_copy(z_hbm, acc)
            for ch in range(E // _CH):
                pltpu.sync_copy(flat_hbm.at[pl.ds(ch * _CH, _CH)], fv)

                def body(g, carry):
                    f16 = fv[pl.ds(g * 16, 16)]
                    lf = f16 - base
                    m = (lf >= 0) & (lf < _HALF * N)
                    lf = jnp.where(m, lf, 0)
                    plsc.addupdate_scatter(acc, [lf], ones16, mask=m)
                    return carry

                lax.fori_loop(0, _CH // 16, body, 0)
            pltpu.sync_copy(acc, out_hbm.at[pl.ds(base, _HALF * N)])

    return k(flat, zblock).reshape(N, N)


_NN = (((1,), (0,)), ((), ()))
_TN = (((0,), (0,)), ((), ()))


def _b(x):
    return x.astype(jnp.bfloat16)


def _bfdot(a, b, dims=_NN):
    return lax.dot_general(_b(a), _b(b), dims,
                           preferred_element_type=jnp.float32)


def _split3(x):
    h1 = _b(x).astype(jnp.float32)
    r1 = x - h1
    h2 = _b(r1).astype(jnp.float32)
    r2 = r1 - h2
    return h1, h2, r2


def _exact_dot(sel, m, dims):
    selb = _b(sel)
    out = None
    for part in _split3(m):
        d = lax.dot_general(selb, _b(part), dims,
                            preferred_element_type=jnp.float32)
        out = d if out is None else out + d
    return out


def _eye(n):
    r = lax.broadcasted_iota(jnp.int32, (n, n), 0)
    c = lax.broadcasted_iota(jnp.int32, (n, n), 1)
    return jnp.where(r == c, 1.0, 0.0).astype(jnp.float32)


def _gcn_body(t_mat, xw, b_row):
    n = t_mat.shape[0]
    ey = _eye(n)
    dvec = jnp.sum(t_mat * ey, axis=1, keepdims=True)
    fill = jnp.where(dvec == 0.0, 2.0, 0.0)
    t2 = t_mat + ey * fill
    deg = jnp.sum(t2, axis=1, keepdims=True)
    dinv = jnp.where(deg > 0.0,
                     1.0 / jnp.sqrt(jnp.maximum(deg, 1e-12)), 0.0)
    ant = (jnp.transpose(dinv, (1, 0)) * t2) * dinv
    return _bfdot(ant, xw) + b_row


def _proj_kernel(x_ref, w_ref, o_ref):
    o_ref[...] = _bfdot(x_ref[...], w_ref[...])


def _gcn0_kernel(t_ref, xw_ref, b_ref, o_ref):
    y = _gcn_body(t_ref[...], xw_ref[...], b_ref[...])
    o_ref[...] = jnp.maximum(y, 0.0)


def _transition_kernel(t_ref, x_ref, pw_ref, w_ref, b_ref,
                       tn_ref, xn_ref, rt_ref, *, n, kk):
    t_prev = t_ref[...]
    x_prev = x_ref[...]
    pw = pw_ref[...]

    ey = _eye(n)
    t_tilde = t_prev * (1.0 - ey) + ey

    s_raw = _bfdot(x_prev, jnp.transpose(pw, (1, 0)))
    s_col = jnp.tanh(s_raw / jnp.sqrt(jnp.sum(pw * pw)))
    s_row = jnp.transpose(s_col, (1, 0))
    i_col = lax.broadcasted_iota(jnp.int32, (n, 1), 0)
    i_row = lax.broadcasted_iota(jnp.int32, (1, n), 1)
    gt = (s_row > s_col) | ((s_row == s_col) & (i_row < i_col))
    rank = jnp.sum(jnp.where(gt, 1.0, 0.0), axis=1, keepdims=True)
    r_row = lax.broadcasted_iota(jnp.int32, (1, kk), 1).astype(jnp.float32)
    rt = jnp.where(rank == r_row, 1.0, 0.0)

    vals = _exact_dot(rt, s_col, _TN)
    xp = _exact_dot(rt, x_prev, _TN) * vals

    tb = _b(t_tilde)
    rtb = _b(rt)
    bsel = lax.dot_general(rtb, tb, _TN,
                           preferred_element_type=jnp.float32)
    csel = lax.dot_general(tb, rtb, _NN,
                           preferred_element_type=jnp.float32)
    c3 = _bfdot(bsel, csel)
    eyk = _eye(kk)
    t_next = c3 * (1.0 - eyk)

    t2 = t_next + 2.0 * eyk
    deg = jnp.sum(t2, axis=1, keepdims=True)
    dinv = jnp.where(deg > 0.0,
                     1.0 / jnp.sqrt(jnp.maximum(deg, 1e-12)), 0.0)
    ant = (jnp.transpose(dinv, (1, 0)) * t2) * dinv
    xw = _bfdot(xp, w_ref[...])
    y = _bfdot(ant, xw) + b_ref[...]

    tn_ref[...] = t_next
    xn_ref[...] = jnp.maximum(y, 0.0)
    rt_ref[...] = rt


def _up_kernel(t_ref, res_ref, x_ref, rt_ref, w_ref, b_ref, o_ref,
               *, last):
    up = _exact_dot(rt_ref[...], x_ref[...], _NN)
    x = res_ref[...] + up
    xw = _bfdot(x, w_ref[...])
    y = _gcn_body(t_ref[...], xw, b_ref[...])
    if last:
        ss = jnp.sum(y * y, axis=1, keepdims=True)
        y = y / jnp.maximum(jnp.sqrt(jnp.maximum(ss, 1e-12)), 1e-3)
    else:
        y = jnp.maximum(y, 0.0)
    o_ref[...] = y


def _call(fn, out_shapes, *args, **static):
    f = functools.partial(fn, **static) if static else fn
    return pl.pallas_call(
        f,
        out_shape=out_shapes,
    )(*args)


def kernel(x, edge_index, params):
    f32 = jnp.float32
    rows = edge_index[0].astype(jnp.int32)
    cols = edge_index[1].astype(jnp.int32)
    zblock = jnp.zeros((_HALF * N,), f32)

    flat = _call(_flat_kernel, jax.ShapeDtypeStruct((E // 128, 128), jnp.int32),
                 rows.reshape(E // 128, 128),
                 cols.reshape(E // 128, 128)).reshape(E)

    xw0 = _call(_proj_kernel, jax.ShapeDtypeStruct((N, 16), f32),
                x, params["W_down"][0])

    t0 = _build_t_sc(flat, zblock)

    b0 = params["b_down"][0].reshape(1, -1)
    x_cur = _call(_gcn0_kernel, jax.ShapeDtypeStruct((N, 16), f32),
                  t0, xw0, b0)

    sizes = [N // (2 ** i) for i in range(DEPTH + 1)]
    ts, xs, rts = [t0], [x_cur], []
    for i in range(1, DEPTH + 1):
        n, kk = sizes[i - 1], sizes[i]
        pw = params["pool_w"][i - 1].reshape(1, -1)
        wd = params["W_down"][i]
        bd = params["b_down"][i].reshape(1, -1)
        outs = (jax.ShapeDtypeStruct((kk, kk), f32),
                jax.ShapeDtypeStruct((kk, 16), f32),
                jax.ShapeDtypeStruct((n, kk), f32))
        t_next, x_cur, rt = _call(_transition_kernel, outs,
                                  ts[-1], xs[-1], pw, wd, bd, n=n, kk=kk)
        ts.append(t_next)
        xs.append(x_cur)
        rts.append(rt)

    for i in range(DEPTH):
        j = DEPTH - 1 - i
        wu = params["W_up"][i]
        bu = params["b_up"][i].reshape(1, -1)
        out_ch = wu.shape[1]
        x_cur = _call(_up_kernel,
                      jax.ShapeDtypeStruct((sizes[j], out_ch), f32),
                      ts[j], xs[j], x_cur, rts[j], wu, bu,
                      last=(i == DEPTH - 1))

    return x_cur
